# Initial kernel scaffold; baseline (speedup 1.0000x reference)
import jax
import jax.numpy as jnp
from jax.experimental import pallas as pl


def kernel(hs_pad, alloW, phone_arc_labels, phoneme_arc_labels):
    raise NotImplementedError("write your pallas kernel here")


# speedup vs baseline: 11.4498x; 11.4498x over previous
import functools

import jax
import jax.numpy as jnp
from jax.experimental import pallas as pl
from jax.experimental.pallas import tpu as pltpu


def _allo_block_kernel(perm_ref, plab_ref, allow_ref, x_ref, out_ref, m_ref, *, num_p):
    @pl.when(pl.program_id(0) == 0)
    def _build_m():
        a_dim = perm_ref.shape[1]
        c_dim = m_ref.shape[0]
        w = jnp.exp(allow_ref[...])
        iota_c = jax.lax.broadcasted_iota(jnp.int32, (c_dim, a_dim), 0)
        u = jnp.where(iota_c == perm_ref[...], 1.0, 0.0).astype(jnp.bfloat16)
        iota_p = jax.lax.broadcasted_iota(jnp.int32, (num_p, a_dim), 0)
        vt = jnp.where(iota_p == plab_ref[...], w, 0.0).astype(jnp.bfloat16)
        m = jax.lax.dot_general(
            u, vt, (((1,), (1,)), ((), ())), preferred_element_type=jnp.float32
        )
        m_ref[...] = m.astype(jnp.bfloat16)

    x = x_ref[...]
    mx = jnp.max(x, axis=1, keepdims=True)
    e = jnp.exp(x - mx)
    z = jnp.sum(e, axis=1, keepdims=True)
    g = jnp.dot(e.astype(jnp.bfloat16), m_ref[...], preferred_element_type=jnp.float32)
    sg = jnp.sum(g, axis=1, keepdims=True)
    out_ref[...] = jnp.log(g - (sg - z) * (1.0 / num_p)) - jnp.log(z)


def kernel(hs_pad, alloW, phone_arc_labels, phoneme_arc_labels):
    b_dim, t_dim, c_dim = hs_pad.shape
    a_dim = alloW.shape[0]
    p_dim = 512
    rows = b_dim * t_dim
    block_r = 512
    grid = (rows // block_r,)

    x2d = hs_pad.reshape(rows, c_dim)
    perm2d = phone_arc_labels.reshape(1, a_dim)
    plab2d = phoneme_arc_labels.reshape(1, a_dim)
    allow2d = alloW.reshape(1, a_dim)

    out = pl.pallas_call(
        functools.partial(_allo_block_kernel, num_p=p_dim),
        grid=grid,
        in_specs=[
            pl.BlockSpec((1, a_dim), lambda i: (0, 0)),
            pl.BlockSpec((1, a_dim), lambda i: (0, 0)),
            pl.BlockSpec((1, a_dim), lambda i: (0, 0)),
            pl.BlockSpec((block_r, c_dim), lambda i: (i, 0)),
        ],
        out_specs=pl.BlockSpec((block_r, p_dim), lambda i: (i, 0)),
        out_shape=jax.ShapeDtypeStruct((rows, p_dim), jnp.float32),
        scratch_shapes=[pltpu.VMEM((c_dim, p_dim), jnp.bfloat16)],
        compiler_params=pltpu.CompilerParams(
            dimension_semantics=("arbitrary",),
        ),
    )(perm2d, plab2d, allow2d, x2d)
    return out.reshape(b_dim, t_dim, p_dim)
